# Initial kernel scaffold; baseline (speedup 1.0000x reference)
import jax
import jax.numpy as jnp
from jax.experimental import pallas as pl


def kernel(inp, target, weights, code_weights):
    raise NotImplementedError("write your pallas kernel here")


# speedup vs baseline: 1.7756x; 1.7756x over previous
import math

import jax
import jax.numpy as jnp
from jax.experimental import pallas as pl
from jax.experimental.pallas import tpu as pltpu

_BETA = 1.0 / 9.0


def _huber_kernel(x_ref, t_ref, w_ref, cw_ref, ec_ref, ew_ref, o_ref, *, beta):
    x = x_ref[...].astype(jnp.float32)
    t = t_ref[...].astype(jnp.float32)
    cw_row = jnp.dot(cw_ref[...], ec_ref[...], preferred_element_type=jnp.float32)
    w_row = jnp.dot(w_ref[...].astype(jnp.float32), ew_ref[...],
                    preferred_element_type=jnp.float32)
    t = jnp.where(t != t, x, t)
    n = jnp.abs((x - t) * cw_row)
    loss = jnp.where(n < beta, (0.5 / beta) * n * n, n - 0.5 * beta)
    o_ref[...] = (loss * w_row).astype(o_ref.dtype)


def _row_tile(rows):
    best = None
    for tr in range(8, rows + 1, 8):
        if rows % tr:
            continue
        steps = rows // tr
        if steps % 2:
            continue
        if best is None or tr <= 384:
            best = tr
    if best is not None:
        return best
    for tr in range(8, rows + 1, 8):
        if rows % tr == 0:
            return tr
    return rows


def kernel(inp, target, weights, code_weights):
    B, A, C = inp.shape
    lanes = C * 128 // math.gcd(C, 128)
    wlanes = lanes // C
    n = A * C
    assert n % lanes == 0 and A % wlanes == 0, "unsupported geometry"
    rows = B * (n // lanes)
    tr = _row_tile(rows)
    grid = rows // tr
    out_dtype = inp.dtype

    x2 = inp.reshape(rows, lanes)
    t2 = target.reshape(rows, lanes)
    w2 = weights.reshape(rows, wlanes)
    cw2 = code_weights.astype(jnp.float32).reshape(1, C)

    lane_c = jax.lax.broadcasted_iota(jnp.int32, (C, lanes), 1)
    code_c = jax.lax.broadcasted_iota(jnp.int32, (C, lanes), 0)
    ec = (lane_c % C == code_c).astype(jnp.float32)
    lane_w = jax.lax.broadcasted_iota(jnp.int32, (wlanes, lanes), 1)
    anchor_w = jax.lax.broadcasted_iota(jnp.int32, (wlanes, lanes), 0)
    ew = (lane_w // C == anchor_w).astype(jnp.float32)

    def body(x_ref, t_ref, w_ref, cw_ref, ec_ref, ew_ref, o_ref):
        _huber_kernel(x_ref, t_ref, w_ref, cw_ref, ec_ref, ew_ref, o_ref,
                      beta=float(_BETA))

    out2 = pl.pallas_call(
        body,
        out_shape=jax.ShapeDtypeStruct((rows, lanes), out_dtype),
        grid=(grid,),
        in_specs=[
            pl.BlockSpec((tr, lanes), lambda i: (i, 0)),
            pl.BlockSpec((tr, lanes), lambda i: (i, 0)),
            pl.BlockSpec((tr, wlanes), lambda i: (i, 0)),
            pl.BlockSpec((1, C), lambda i: (0, 0)),
            pl.BlockSpec((C, lanes), lambda i: (0, 0)),
            pl.BlockSpec((wlanes, lanes), lambda i: (0, 0)),
        ],
        out_specs=pl.BlockSpec((tr, lanes), lambda i: (i, 0)),
        compiler_params=pltpu.CompilerParams(
            dimension_semantics=("parallel",)),
    )(x2, t2, w2, cw2, ec, ew)

    return out2.reshape(B, A, C)


# speedup vs baseline: 54.6509x; 30.7786x over previous
import jax
import jax.numpy as jnp
from jax.experimental import pallas as pl
from jax.experimental.pallas import tpu as pltpu

_BETA = 1.0 / 9.0


def _huber_kernel(cw_sref, x_ref, t_ref, w_ref, o_ref, *, beta):
    c = pl.program_id(1)
    cw = cw_sref[c]
    x = x_ref[...].astype(jnp.float32)
    t = t_ref[...].astype(jnp.float32)
    t = jnp.where(t != t, x, t)
    n = jnp.abs((x - t) * cw)
    loss = jnp.where(n < beta, (0.5 / beta) * n * n, n - 0.5 * beta)
    o_ref[...] = (loss * w_ref[...].astype(jnp.float32)).astype(o_ref.dtype)


def _anchor_tile(a):
    best = None
    for la in range(128, a + 1, 128):
        if a % la:
            continue
        if (a // la) % 2:
            continue
        if best is None or la <= 12288:
            best = la
    if best is not None:
        return best
    for la in range(128, a + 1, 128):
        if a % la == 0:
            return la
    return a


def kernel(inp, target, weights, code_weights):
    B, A, C = inp.shape
    assert A % 256 == 0, "unsupported geometry"
    la = _anchor_tile(A)
    ka = A // la
    out_dtype = inp.dtype

    x3 = jnp.transpose(inp, (2, 0, 1))
    t3 = jnp.transpose(target, (2, 0, 1))
    cw = code_weights.astype(jnp.float32)

    out3 = pl.pallas_call(
        lambda s, x, t, w, o: _huber_kernel(s, x, t, w, o, beta=float(_BETA)),
        out_shape=jax.ShapeDtypeStruct((C, B, A), out_dtype),
        grid_spec=pltpu.PrefetchScalarGridSpec(
            num_scalar_prefetch=1,
            grid=(ka, C),
            in_specs=[
                pl.BlockSpec((1, B, la), lambda a, c, *_: (c, 0, a)),
                pl.BlockSpec((1, B, la), lambda a, c, *_: (c, 0, a)),
                pl.BlockSpec((B, la), lambda a, c, *_: (0, a)),
            ],
            out_specs=pl.BlockSpec((1, B, la), lambda a, c, *_: (c, 0, a)),
        ),
        compiler_params=pltpu.CompilerParams(
            dimension_semantics=("parallel", "arbitrary")),
    )(cw, x3, t3, weights)

    return jnp.transpose(out3, (1, 2, 0))


# speedup vs baseline: 74.9835x; 1.3720x over previous
import jax
import jax.numpy as jnp
from jax.experimental import pallas as pl
from jax.experimental.pallas import tpu as pltpu

_BETA = 1.0 / 9.0


def _huber_kernel(cw_sref, x_ref, t_ref, w_ref, o_ref, *, beta):
    c = pl.program_id(1)
    cw = cw_sref[c]
    x = x_ref[...].astype(jnp.float32)
    t = t_ref[...].astype(jnp.float32)
    t = jnp.where(t != t, x, t)
    n = jnp.abs((x - t) * cw)
    loss = jnp.where(n < beta, (0.5 / beta) * n * n, n - 0.5 * beta)
    o_ref[...] = (loss * w_ref[...].astype(jnp.float32)).astype(o_ref.dtype)


def _anchor_tile(a):
    best = None
    for la in range(128, a + 1, 128):
        if a % la:
            continue
        if (a // la) % 2:
            continue
        if best is None or la <= 34944:
            best = la
    if best is not None:
        return best
    for la in range(128, a + 1, 128):
        if a % la == 0:
            return la
    return a


def kernel(inp, target, weights, code_weights):
    B, A, C = inp.shape
    assert A % 256 == 0, "unsupported geometry"
    la = _anchor_tile(A)
    ka = A // la
    out_dtype = inp.dtype

    x3 = jnp.transpose(inp, (2, 0, 1))
    t3 = jnp.transpose(target, (2, 0, 1))
    cw = code_weights.astype(jnp.float32)

    out3 = pl.pallas_call(
        lambda s, x, t, w, o: _huber_kernel(s, x, t, w, o, beta=float(_BETA)),
        out_shape=jax.ShapeDtypeStruct((C, B, A), out_dtype),
        grid_spec=pltpu.PrefetchScalarGridSpec(
            num_scalar_prefetch=1,
            grid=(ka, C),
            in_specs=[
                pl.BlockSpec((1, B, la), lambda a, c, *_: (c, 0, a)),
                pl.BlockSpec((1, B, la), lambda a, c, *_: (c, 0, a)),
                pl.BlockSpec((B, la), lambda a, c, *_: (0, a)),
            ],
            out_specs=pl.BlockSpec((1, B, la), lambda a, c, *_: (c, 0, a)),
        ),
        compiler_params=pltpu.CompilerParams(
            dimension_semantics=("parallel", "arbitrary")),
    )(cw, x3, t3, weights)

    return jnp.transpose(out3, (1, 2, 0))
